# baseline (speedup 1.0000x reference)
import jax
import jax.numpy as jnp
from jax import lax
from jax.experimental import pallas as pl
from jax.experimental.pallas import tpu as pltpu
from jax.experimental.pallas import tpu_sc as plsc

N = 10000
E = 320000
D = 128
STEPS = 5

NC, NS = 2, 16
NW = NC * NS
CHUNK = 128
NCHUNK = 80
WCH = 8
NWIN = NCHUNK // WCH
E_PAD = NW * NCHUNK * CHUNK
ACC_ROWS = 10112
ROWS_PER_TILE = ACC_ROWS // NS


def _sc_body(wh_hbm, src_hbm, dst_hbm, out_hbm,
             sw0, dw0, sw1, dw1, rows_a, rows_b, acc_sh,
             gsa, gsb, ws0, ws1, ssa, ssb):
    c = lax.axis_index("c")
    s = lax.axis_index("s")
    wid = s * NC + c
    wbase = wid * NWIN

    def zfill(i, _):
        r = i // (D // 16)
        k = i % (D // 16)
        rows_a[r, pl.ds(k * 16, 16)] = jnp.zeros((16,), jnp.float32)
        return 0
    lax.fori_loop(0, CHUNK * (D // 16), zfill, 0)
    base = s * ROWS_PER_TILE
    for j in range(ROWS_PER_TILE // CHUNK):
        pltpu.sync_copy(rows_a, acc_sh.at[pl.ds(base + j * CHUNK, CHUNK)])
    rem = ROWS_PER_TILE % CHUNK
    if rem:
        pltpu.sync_copy(
            rows_a.at[pl.ds(0, rem)],
            acc_sh.at[pl.ds(base + (ROWS_PER_TILE // CHUNK) * CHUNK, rem)])
    plsc.subcore_barrier()

    rows = (rows_a, rows_b)
    gs = (gsa, gsb)

    def idx_load(w, sb, db, sem):
        pltpu.async_copy(src_hbm.at[wbase + w], sb, sem)
        pltpu.async_copy(dst_hbm.at[wbase + w], db, sem)

    def idx_wait(sb, db, sem):
        pltpu.make_async_copy(src_hbm.at[wbase], sb, sem).wait()
        pltpu.make_async_copy(dst_hbm.at[wbase], db, sem).wait()

    ss = (ssa, ssb)

    def gather(sb, j, par):
        pltpu.async_copy(wh_hbm.at[sb.at[j]], rows[par], gs[par])

    def gather_wait(par):
        pltpu.make_async_copy(wh_hbm.at[sw0.at[0]], rows[par], gs[par]).wait()

    def scat_start(db, j, par):
        pltpu.async_copy(rows[par], acc_sh.at[db.at[j]], ss[par], add=True)

    def scat_wait(par):
        pltpu.make_async_copy(rows[par], acc_sh.at[dw0.at[0]], ss[par]).wait()

    def process_window(sb, db, nsb, ndb, nws, prefetch_next, first=False):
        for j in range(WCH):
            par = j % 2
            gather_wait(par)
            scat_start(db, j, par)
            if j + 1 < WCH:
                if not (first and j == 0):
                    scat_wait(1 - par)
                gather(sb, j + 1, 1 - par)
            elif prefetch_next:
                idx_wait(nsb, ndb, nws)
                scat_wait(1 - par)
                gather(nsb, 0, 1 - par)

    idx_load(0, sw0, dw0, ws0)
    idx_load(1, sw1, dw1, ws1)
    idx_wait(sw0, dw0, ws0)
    gather(sw0, 0, 0)
    process_window(sw0, dw0, sw1, dw1, ws1, True, first=True)
    idx_load(2, sw0, dw0, ws0)

    def pair(p, _):
        process_window(sw1, dw1, sw0, dw0, ws0, True)
        idx_load(2 * p + 3, sw1, dw1, ws1)
        process_window(sw0, dw0, sw1, dw1, ws1, True)
        idx_load(2 * p + 4, sw0, dw0, ws0)
        return 0
    lax.fori_loop(0, NWIN // 2 - 2, pair, 0)

    process_window(sw1, dw1, sw0, dw0, ws0, True)
    idx_load(NWIN - 1, sw1, dw1, ws1)
    process_window(sw0, dw0, sw1, dw1, ws1, True)
    process_window(sw1, dw1, None, None, None, False)
    scat_wait(0)
    scat_wait(1)
    plsc.subcore_barrier()

    pltpu.sync_copy(acc_sh.at[pl.ds(base, ROWS_PER_TILE)],
                    out_hbm.at[pl.ds(c * ACC_ROWS + base, ROWS_PER_TILE)])


_sc_scatter = pl.kernel(
    _sc_body,
    out_type=jax.ShapeDtypeStruct((NC * ACC_ROWS, D), jnp.float32),
    mesh=plsc.VectorSubcoreMesh(core_axis_name="c", subcore_axis_name="s"),
    scratch_types=[
        pltpu.VMEM((WCH, CHUNK), jnp.int32),
        pltpu.VMEM((WCH, CHUNK), jnp.int32),
        pltpu.VMEM((WCH, CHUNK), jnp.int32),
        pltpu.VMEM((WCH, CHUNK), jnp.int32),
        pltpu.VMEM((CHUNK, D), jnp.float32),
        pltpu.VMEM((CHUNK, D), jnp.float32),
        pltpu.VMEM_SHARED((ACC_ROWS, D), jnp.float32),
        pltpu.SemaphoreType.DMA,
        pltpu.SemaphoreType.DMA,
        pltpu.SemaphoreType.DMA,
        pltpu.SemaphoreType.DMA,
        pltpu.SemaphoreType.DMA,
        pltpu.SemaphoreType.DMA,
    ],
)


ROW_BLK = 1000


def _linear_body(x_ref, wT_ref, b_ref, out_ref):
    out_ref[...] = (
        jnp.dot(x_ref[...], wT_ref[...], preferred_element_type=jnp.float32)
        + b_ref[...]
    )


def _linear(x, WT, b2):
    return pl.pallas_call(
        _linear_body,
        grid=(N // ROW_BLK,),
        in_specs=[
            pl.BlockSpec((ROW_BLK, D), lambda i: (i, 0)),
            pl.BlockSpec((D, D), lambda i: (0, 0)),
            pl.BlockSpec((1, D), lambda i: (0, 0)),
        ],
        out_specs=pl.BlockSpec((ROW_BLK, D), lambda i: (i, 0)),
        out_shape=jax.ShapeDtypeStruct((N, D), jnp.float32),
    )(x, WT, b2)


def _gru_body(a0_ref, a1_ref, h_ref, wihT_ref, whhT_ref, bih_ref, bhh_ref,
              wT_ref, b_ref, h_out, wh_out):
    a = a0_ref[...] + a1_ref[...]
    h = h_ref[...]
    gi = jnp.dot(a, wihT_ref[...], preferred_element_type=jnp.float32) + bih_ref[...]
    gh = jnp.dot(h, whhT_ref[...], preferred_element_type=jnp.float32) + bhh_ref[...]
    r = jax.nn.sigmoid(gi[:, :D] + gh[:, :D])
    z = jax.nn.sigmoid(gi[:, D:2 * D] + gh[:, D:2 * D])
    n = jnp.tanh(gi[:, 2 * D:] + r * gh[:, 2 * D:])
    hn = (1.0 - z) * n + z * h
    h_out[...] = hn
    wh_out[...] = (
        jnp.dot(hn, wT_ref[...], preferred_element_type=jnp.float32) + b_ref[...]
    )


def _gru_step(a0, a1, h, WihT, WhhT, bih2, bhh2, WT, b2):
    return pl.pallas_call(
        _gru_body,
        grid=(N // ROW_BLK,),
        in_specs=[
            pl.BlockSpec((ROW_BLK, D), lambda i: (i, 0)),
            pl.BlockSpec((ROW_BLK, D), lambda i: (i, 0)),
            pl.BlockSpec((ROW_BLK, D), lambda i: (i, 0)),
            pl.BlockSpec((D, 3 * D), lambda i: (0, 0)),
            pl.BlockSpec((D, 3 * D), lambda i: (0, 0)),
            pl.BlockSpec((1, 3 * D), lambda i: (0, 0)),
            pl.BlockSpec((1, 3 * D), lambda i: (0, 0)),
            pl.BlockSpec((D, D), lambda i: (0, 0)),
            pl.BlockSpec((1, D), lambda i: (0, 0)),
        ],
        out_specs=[
            pl.BlockSpec((ROW_BLK, D), lambda i: (i, 0)),
            pl.BlockSpec((ROW_BLK, D), lambda i: (i, 0)),
        ],
        out_shape=[
            jax.ShapeDtypeStruct((N, D), jnp.float32),
            jax.ShapeDtypeStruct((N, D), jnp.float32),
        ],
    )(a0, a1, h, WihT, WhhT, bih2, bhh2, WT, b2)


@jax.jit
def kernel(x, edge_index, W, b, W_ih, W_hh, b_ih, b_hh):
    WT = W.T
    WihT = W_ih.T
    WhhT = W_hh.T
    b2 = b.reshape(1, D)
    bih2 = b_ih.reshape(1, 3 * D)
    bhh2 = b_hh.reshape(1, 3 * D)

    pad = E_PAD - E
    it = jnp.arange(pad, dtype=jnp.int32)
    src = jnp.concatenate([edge_index[0], (it * 97) % N])
    dst = jnp.concatenate([edge_index[1], N + (it % (ACC_ROWS - N))])
    src3 = (src.reshape(NCHUNK, NW, CHUNK).transpose(1, 0, 2)
            .reshape(NW * NWIN, WCH, CHUNK))
    dst3 = (dst.reshape(NCHUNK, NW, CHUNK).transpose(1, 0, 2)
            .reshape(NW * NWIN, WCH, CHUNK))

    h = x
    Wh = _linear(x, WT, b2)
    for _ in range(STEPS):
        parts = _sc_scatter(Wh, src3, dst3)
        a0 = parts[:N]
        a1 = parts[ACC_ROWS:ACC_ROWS + N]
        h, Wh = _gru_step(a0, a1, h, WihT, WhhT, bih2, bhh2, WT, b2)
    return h


# speedup vs baseline: 1.0011x; 1.0011x over previous
import jax
import jax.numpy as jnp
from jax import lax
from jax.experimental import pallas as pl
from jax.experimental.pallas import tpu as pltpu
from jax.experimental.pallas import tpu_sc as plsc

N = 10000
E = 320000
D = 128
STEPS = 5

NC, NS = 2, 16
NW = NC * NS
CHUNK = 128
NCHUNK = 80
WCH = 8
NWIN = NCHUNK // WCH
E_PAD = NW * NCHUNK * CHUNK
ACC_ROWS = 10112
ROWS_PER_TILE = ACC_ROWS // NS


def _sc_body(wh_hbm, src_hbm, dst_hbm, out_hbm,
             sw0, dw0, sw1, dw1, rows_a, rows_b, acc_sh,
             gsa, gsb, ws0, ws1, ssa, ssb):
    c = lax.axis_index("c")
    s = lax.axis_index("s")
    wid = s * NC + c
    wbase = wid * NWIN

    def zfill(i, _):
        r = i // (D // 16)
        k = i % (D // 16)
        rows_a[r, pl.ds(k * 16, 16)] = jnp.zeros((16,), jnp.float32)
        return 0
    lax.fori_loop(0, CHUNK * (D // 16), zfill, 0)
    base = s * ROWS_PER_TILE
    for j in range(ROWS_PER_TILE // CHUNK):
        pltpu.sync_copy(rows_a, acc_sh.at[pl.ds(base + j * CHUNK, CHUNK)])
    rem = ROWS_PER_TILE % CHUNK
    if rem:
        pltpu.sync_copy(
            rows_a.at[pl.ds(0, rem)],
            acc_sh.at[pl.ds(base + (ROWS_PER_TILE // CHUNK) * CHUNK, rem)])
    plsc.subcore_barrier()

    rows = (rows_a, rows_b)
    gs = (gsa, gsb)

    def idx_load(w, sb, db, sem):
        pltpu.async_copy(src_hbm.at[wbase + w], sb, sem)
        pltpu.async_copy(dst_hbm.at[wbase + w], db, sem)

    def idx_wait(sb, db, sem):
        pltpu.make_async_copy(src_hbm.at[wbase], sb, sem).wait()
        pltpu.make_async_copy(dst_hbm.at[wbase], db, sem).wait()

    ss = (ssa, ssb)

    def gather(sb, j, par):
        pltpu.async_copy(wh_hbm.at[sb.at[j]], rows[par], gs[par])

    def gather_wait(par):
        pltpu.make_async_copy(wh_hbm.at[sw0.at[0]], rows[par], gs[par]).wait()

    def scat_start(db, j, par):
        pltpu.async_copy(rows[par], acc_sh.at[db.at[j]], ss[par], add=True)

    def scat_wait(par):
        pltpu.make_async_copy(rows[par], acc_sh.at[dw0.at[0]], ss[par]).wait()

    def process_window(sb, db, nsb, ndb, nws, prefetch_next, first=False):
        for j in range(WCH):
            par = j % 2
            gather_wait(par)
            scat_start(db, j, par)
            if j + 1 < WCH:
                if not (first and j == 0):
                    scat_wait(1 - par)
                gather(sb, j + 1, 1 - par)
            elif prefetch_next:
                idx_wait(nsb, ndb, nws)
                scat_wait(1 - par)
                gather(nsb, 0, 1 - par)

    idx_load(0, sw0, dw0, ws0)
    idx_load(1, sw1, dw1, ws1)
    idx_wait(sw0, dw0, ws0)
    gather(sw0, 0, 0)
    process_window(sw0, dw0, sw1, dw1, ws1, True, first=True)
    idx_load(2, sw0, dw0, ws0)

    def pair(p, _):
        process_window(sw1, dw1, sw0, dw0, ws0, True)
        idx_load(2 * p + 3, sw1, dw1, ws1)
        process_window(sw0, dw0, sw1, dw1, ws1, True)
        idx_load(2 * p + 4, sw0, dw0, ws0)
        return 0
    lax.fori_loop(0, NWIN // 2 - 2, pair, 0)

    process_window(sw1, dw1, sw0, dw0, ws0, True)
    idx_load(NWIN - 1, sw1, dw1, ws1)
    process_window(sw0, dw0, sw1, dw1, ws1, True)
    process_window(sw1, dw1, None, None, None, False)
    scat_wait(0)
    scat_wait(1)
    plsc.subcore_barrier()

    pltpu.sync_copy(acc_sh.at[pl.ds(base, ROWS_PER_TILE)],
                    out_hbm.at[pl.ds(c * ACC_ROWS + base, ROWS_PER_TILE)])


_sc_scatter = pl.kernel(
    _sc_body,
    out_type=jax.ShapeDtypeStruct((NC * ACC_ROWS, D), jnp.float32),
    mesh=plsc.VectorSubcoreMesh(core_axis_name="c", subcore_axis_name="s"),
    scratch_types=[
        pltpu.VMEM((WCH, CHUNK), jnp.int32),
        pltpu.VMEM((WCH, CHUNK), jnp.int32),
        pltpu.VMEM((WCH, CHUNK), jnp.int32),
        pltpu.VMEM((WCH, CHUNK), jnp.int32),
        pltpu.VMEM((CHUNK, D), jnp.float32),
        pltpu.VMEM((CHUNK, D), jnp.float32),
        pltpu.VMEM_SHARED((ACC_ROWS, D), jnp.float32),
        pltpu.SemaphoreType.DMA,
        pltpu.SemaphoreType.DMA,
        pltpu.SemaphoreType.DMA,
        pltpu.SemaphoreType.DMA,
        pltpu.SemaphoreType.DMA,
        pltpu.SemaphoreType.DMA,
    ],
)


ROW_BLK = 1000


def _linear_body(x_ref, wT_ref, b_ref, out_ref):
    out_ref[...] = (
        jnp.dot(x_ref[...], wT_ref[...], preferred_element_type=jnp.float32)
        + b_ref[...]
    )


def _linear(x, WT, b2):
    return pl.pallas_call(
        _linear_body,
        grid=(N // ROW_BLK,),
        in_specs=[
            pl.BlockSpec((ROW_BLK, D), lambda i: (i, 0)),
            pl.BlockSpec((D, D), lambda i: (0, 0)),
            pl.BlockSpec((1, D), lambda i: (0, 0)),
        ],
        out_specs=pl.BlockSpec((ROW_BLK, D), lambda i: (i, 0)),
        out_shape=jax.ShapeDtypeStruct((N, D), jnp.float32),
    )(x, WT, b2)


def _gru_body(a0_ref, a1_ref, h_ref, wihT_ref, whhT_ref, bih_ref, bhh_ref,
              wT_ref, b_ref, h_out, wh_out):
    a = a0_ref[...] + a1_ref[...]
    h = h_ref[...]
    gi = jnp.dot(a, wihT_ref[...], preferred_element_type=jnp.float32) + bih_ref[...]
    gh = jnp.dot(h, whhT_ref[...], preferred_element_type=jnp.float32) + bhh_ref[...]
    r = jax.nn.sigmoid(gi[:, :D] + gh[:, :D])
    z = jax.nn.sigmoid(gi[:, D:2 * D] + gh[:, D:2 * D])
    n = jnp.tanh(gi[:, 2 * D:] + r * gh[:, 2 * D:])
    hn = (1.0 - z) * n + z * h
    h_out[...] = hn
    wh_out[...] = (
        jnp.dot(hn, wT_ref[...], preferred_element_type=jnp.float32) + b_ref[...]
    )


def _gru_last_body(a0_ref, a1_ref, h_ref, wihT_ref, whhT_ref, bih_ref,
                   bhh_ref, h_out):
    a = a0_ref[...] + a1_ref[...]
    h = h_ref[...]
    gi = jnp.dot(a, wihT_ref[...], preferred_element_type=jnp.float32) + bih_ref[...]
    gh = jnp.dot(h, whhT_ref[...], preferred_element_type=jnp.float32) + bhh_ref[...]
    r = jax.nn.sigmoid(gi[:, :D] + gh[:, :D])
    z = jax.nn.sigmoid(gi[:, D:2 * D] + gh[:, D:2 * D])
    n = jnp.tanh(gi[:, 2 * D:] + r * gh[:, 2 * D:])
    h_out[...] = (1.0 - z) * n + z * h


def _gru_last(a0, a1, h, WihT, WhhT, bih2, bhh2):
    return pl.pallas_call(
        _gru_last_body,
        grid=(N // ROW_BLK,),
        in_specs=[
            pl.BlockSpec((ROW_BLK, D), lambda i: (i, 0)),
            pl.BlockSpec((ROW_BLK, D), lambda i: (i, 0)),
            pl.BlockSpec((ROW_BLK, D), lambda i: (i, 0)),
            pl.BlockSpec((D, 3 * D), lambda i: (0, 0)),
            pl.BlockSpec((D, 3 * D), lambda i: (0, 0)),
            pl.BlockSpec((1, 3 * D), lambda i: (0, 0)),
            pl.BlockSpec((1, 3 * D), lambda i: (0, 0)),
        ],
        out_specs=pl.BlockSpec((ROW_BLK, D), lambda i: (i, 0)),
        out_shape=jax.ShapeDtypeStruct((N, D), jnp.float32),
    )(a0, a1, h, WihT, WhhT, bih2, bhh2)


def _gru_step(a0, a1, h, WihT, WhhT, bih2, bhh2, WT, b2):
    return pl.pallas_call(
        _gru_body,
        grid=(N // ROW_BLK,),
        in_specs=[
            pl.BlockSpec((ROW_BLK, D), lambda i: (i, 0)),
            pl.BlockSpec((ROW_BLK, D), lambda i: (i, 0)),
            pl.BlockSpec((ROW_BLK, D), lambda i: (i, 0)),
            pl.BlockSpec((D, 3 * D), lambda i: (0, 0)),
            pl.BlockSpec((D, 3 * D), lambda i: (0, 0)),
            pl.BlockSpec((1, 3 * D), lambda i: (0, 0)),
            pl.BlockSpec((1, 3 * D), lambda i: (0, 0)),
            pl.BlockSpec((D, D), lambda i: (0, 0)),
            pl.BlockSpec((1, D), lambda i: (0, 0)),
        ],
        out_specs=[
            pl.BlockSpec((ROW_BLK, D), lambda i: (i, 0)),
            pl.BlockSpec((ROW_BLK, D), lambda i: (i, 0)),
        ],
        out_shape=[
            jax.ShapeDtypeStruct((N, D), jnp.float32),
            jax.ShapeDtypeStruct((N, D), jnp.float32),
        ],
    )(a0, a1, h, WihT, WhhT, bih2, bhh2, WT, b2)


@jax.jit
def kernel(x, edge_index, W, b, W_ih, W_hh, b_ih, b_hh):
    WT = W.T
    WihT = W_ih.T
    WhhT = W_hh.T
    b2 = b.reshape(1, D)
    bih2 = b_ih.reshape(1, 3 * D)
    bhh2 = b_hh.reshape(1, 3 * D)

    pad = E_PAD - E
    it = jnp.arange(pad, dtype=jnp.int32)
    src = jnp.concatenate([edge_index[0], (it * 97) % N])
    dst = jnp.concatenate([edge_index[1], N + (it % (ACC_ROWS - N))])
    src3 = (src.reshape(NCHUNK, NW, CHUNK).transpose(1, 0, 2)
            .reshape(NW * NWIN, WCH, CHUNK))
    dst3 = (dst.reshape(NCHUNK, NW, CHUNK).transpose(1, 0, 2)
            .reshape(NW * NWIN, WCH, CHUNK))

    h = x
    Wh = _linear(x, WT, b2)
    for step in range(STEPS):
        parts = _sc_scatter(Wh, src3, dst3)
        a0 = parts[:N]
        a1 = parts[ACC_ROWS:ACC_ROWS + N]
        if step + 1 < STEPS:
            h, Wh = _gru_step(a0, a1, h, WihT, WhhT, bih2, bhh2, WT, b2)
        else:
            h = _gru_last(a0, a1, h, WihT, WhhT, bih2, bhh2)
    return h


# speedup vs baseline: 1.0080x; 1.0070x over previous
import jax
import jax.numpy as jnp
from jax import lax
from jax.experimental import pallas as pl
from jax.experimental.pallas import tpu as pltpu
from jax.experimental.pallas import tpu_sc as plsc

N = 10000
E = 320000
D = 128
STEPS = 5

NC, NS = 2, 16
NW = NC * NS
CHUNK = 128
NCHUNK = 80
WCH = 8
NWIN = NCHUNK // WCH
E_PAD = NW * NCHUNK * CHUNK
ACC_ROWS = 10112
ROWS_PER_TILE = ACC_ROWS // NS


def _sc_body(wh_hbm, src_hbm, dst_hbm, out_hbm,
             sw0, dw0, sw1, dw1, rows_a, rows_b, acc_sh,
             gsa, gsb, ws0, ws1):
    c = lax.axis_index("c")
    s = lax.axis_index("s")
    wid = s * NC + c
    wbase = wid * NWIN

    def zfill(i, _):
        r = i // (D // 16)
        k = i % (D // 16)
        rows_a[r, pl.ds(k * 16, 16)] = jnp.zeros((16,), jnp.float32)
        return 0
    lax.fori_loop(0, CHUNK * (D // 16), zfill, 0)
    base = s * ROWS_PER_TILE
    for j in range(ROWS_PER_TILE // CHUNK):
        pltpu.sync_copy(rows_a, acc_sh.at[pl.ds(base + j * CHUNK, CHUNK)])
    rem = ROWS_PER_TILE % CHUNK
    if rem:
        pltpu.sync_copy(
            rows_a.at[pl.ds(0, rem)],
            acc_sh.at[pl.ds(base + (ROWS_PER_TILE // CHUNK) * CHUNK, rem)])
    plsc.subcore_barrier()

    rows = (rows_a, rows_b)
    gs = (gsa, gsb)

    def idx_load(w, sb, db, sem):
        pltpu.async_copy(src_hbm.at[wbase + w], sb, sem)
        pltpu.async_copy(dst_hbm.at[wbase + w], db, sem)

    def idx_wait(sb, db, sem):
        pltpu.make_async_copy(src_hbm.at[wbase], sb, sem).wait()
        pltpu.make_async_copy(dst_hbm.at[wbase], db, sem).wait()

    def gather(sb, j, par):
        pltpu.async_copy(wh_hbm.at[sb.at[j]], rows[par], gs[par])

    def gather_wait(par):
        pltpu.make_async_copy(wh_hbm.at[sw0.at[0]], rows[par], gs[par]).wait()

    def scat(db, j, par):
        pltpu.sync_copy(rows[par], acc_sh.at[db.at[j]], add=True)

    def process_window(sb, db, nsb, ndb, nws, prefetch_next):
        for j in range(WCH):
            par = j % 2
            gather_wait(par)
            if j + 1 < WCH:
                gather(sb, j + 1, 1 - par)
            elif prefetch_next:
                idx_wait(nsb, ndb, nws)
                gather(nsb, 0, 1 - par)
            scat(db, j, par)

    idx_load(0, sw0, dw0, ws0)
    idx_load(1, sw1, dw1, ws1)
    idx_wait(sw0, dw0, ws0)
    gather(sw0, 0, 0)

    def pair(p, _):
        process_window(sw0, dw0, sw1, dw1, ws1, True)
        idx_load(2 * p + 2, sw0, dw0, ws0)
        process_window(sw1, dw1, sw0, dw0, ws0, True)
        idx_load(2 * p + 3, sw1, dw1, ws1)
        return 0
    lax.fori_loop(0, NWIN // 2 - 1, pair, 0)

    process_window(sw0, dw0, sw1, dw1, ws1, True)
    process_window(sw1, dw1, None, None, None, False)
    plsc.subcore_barrier()

    pltpu.sync_copy(acc_sh.at[pl.ds(base, ROWS_PER_TILE)],
                    out_hbm.at[pl.ds(c * ACC_ROWS + base, ROWS_PER_TILE)])


_sc_scatter = pl.kernel(
    _sc_body,
    out_type=jax.ShapeDtypeStruct((NC * ACC_ROWS, D), jnp.float32),
    mesh=plsc.VectorSubcoreMesh(core_axis_name="c", subcore_axis_name="s"),
    scratch_types=[
        pltpu.VMEM((WCH, CHUNK), jnp.int32),
        pltpu.VMEM((WCH, CHUNK), jnp.int32),
        pltpu.VMEM((WCH, CHUNK), jnp.int32),
        pltpu.VMEM((WCH, CHUNK), jnp.int32),
        pltpu.VMEM((CHUNK, D), jnp.float32),
        pltpu.VMEM((CHUNK, D), jnp.float32),
        pltpu.VMEM_SHARED((ACC_ROWS, D), jnp.float32),
        pltpu.SemaphoreType.DMA,
        pltpu.SemaphoreType.DMA,
        pltpu.SemaphoreType.DMA,
        pltpu.SemaphoreType.DMA,
    ],
)


ROW_BLK = 1000


def _linear_body(x_ref, wT_ref, b_ref, out_ref):
    out_ref[...] = (
        jnp.dot(x_ref[...], wT_ref[...], preferred_element_type=jnp.float32)
        + b_ref[...]
    )


def _linear(x, WT, b2):
    return pl.pallas_call(
        _linear_body,
        grid=(N // ROW_BLK,),
        in_specs=[
            pl.BlockSpec((ROW_BLK, D), lambda i: (i, 0)),
            pl.BlockSpec((D, D), lambda i: (0, 0)),
            pl.BlockSpec((1, D), lambda i: (0, 0)),
        ],
        out_specs=pl.BlockSpec((ROW_BLK, D), lambda i: (i, 0)),
        out_shape=jax.ShapeDtypeStruct((N, D), jnp.float32),
    )(x, WT, b2)


def _gru_body(a0_ref, a1_ref, h_ref, wihT_ref, whhT_ref, bih_ref, bhh_ref,
              wT_ref, b_ref, h_out, wh_out):
    a = a0_ref[...] + a1_ref[...]
    h = h_ref[...]
    gi = jnp.dot(a, wihT_ref[...], preferred_element_type=jnp.float32) + bih_ref[...]
    gh = jnp.dot(h, whhT_ref[...], preferred_element_type=jnp.float32) + bhh_ref[...]
    r = jax.nn.sigmoid(gi[:, :D] + gh[:, :D])
    z = jax.nn.sigmoid(gi[:, D:2 * D] + gh[:, D:2 * D])
    n = jnp.tanh(gi[:, 2 * D:] + r * gh[:, 2 * D:])
    hn = (1.0 - z) * n + z * h
    h_out[...] = hn
    wh_out[...] = (
        jnp.dot(hn, wT_ref[...], preferred_element_type=jnp.float32) + b_ref[...]
    )


def _gru_last_body(a0_ref, a1_ref, h_ref, wihT_ref, whhT_ref, bih_ref,
                   bhh_ref, h_out):
    a = a0_ref[...] + a1_ref[...]
    h = h_ref[...]
    gi = jnp.dot(a, wihT_ref[...], preferred_element_type=jnp.float32) + bih_ref[...]
    gh = jnp.dot(h, whhT_ref[...], preferred_element_type=jnp.float32) + bhh_ref[...]
    r = jax.nn.sigmoid(gi[:, :D] + gh[:, :D])
    z = jax.nn.sigmoid(gi[:, D:2 * D] + gh[:, D:2 * D])
    n = jnp.tanh(gi[:, 2 * D:] + r * gh[:, 2 * D:])
    h_out[...] = (1.0 - z) * n + z * h


def _gru_last(a0, a1, h, WihT, WhhT, bih2, bhh2):
    return pl.pallas_call(
        _gru_last_body,
        grid=(N // ROW_BLK,),
        in_specs=[
            pl.BlockSpec((ROW_BLK, D), lambda i: (i, 0)),
            pl.BlockSpec((ROW_BLK, D), lambda i: (i, 0)),
            pl.BlockSpec((ROW_BLK, D), lambda i: (i, 0)),
            pl.BlockSpec((D, 3 * D), lambda i: (0, 0)),
            pl.BlockSpec((D, 3 * D), lambda i: (0, 0)),
            pl.BlockSpec((1, 3 * D), lambda i: (0, 0)),
            pl.BlockSpec((1, 3 * D), lambda i: (0, 0)),
        ],
        out_specs=pl.BlockSpec((ROW_BLK, D), lambda i: (i, 0)),
        out_shape=jax.ShapeDtypeStruct((N, D), jnp.float32),
    )(a0, a1, h, WihT, WhhT, bih2, bhh2)


def _gru_step(a0, a1, h, WihT, WhhT, bih2, bhh2, WT, b2):
    return pl.pallas_call(
        _gru_body,
        grid=(N // ROW_BLK,),
        in_specs=[
            pl.BlockSpec((ROW_BLK, D), lambda i: (i, 0)),
            pl.BlockSpec((ROW_BLK, D), lambda i: (i, 0)),
            pl.BlockSpec((ROW_BLK, D), lambda i: (i, 0)),
            pl.BlockSpec((D, 3 * D), lambda i: (0, 0)),
            pl.BlockSpec((D, 3 * D), lambda i: (0, 0)),
            pl.BlockSpec((1, 3 * D), lambda i: (0, 0)),
            pl.BlockSpec((1, 3 * D), lambda i: (0, 0)),
            pl.BlockSpec((D, D), lambda i: (0, 0)),
            pl.BlockSpec((1, D), lambda i: (0, 0)),
        ],
        out_specs=[
            pl.BlockSpec((ROW_BLK, D), lambda i: (i, 0)),
            pl.BlockSpec((ROW_BLK, D), lambda i: (i, 0)),
        ],
        out_shape=[
            jax.ShapeDtypeStruct((N, D), jnp.float32),
            jax.ShapeDtypeStruct((N, D), jnp.float32),
        ],
    )(a0, a1, h, WihT, WhhT, bih2, bhh2, WT, b2)


@jax.jit
def kernel(x, edge_index, W, b, W_ih, W_hh, b_ih, b_hh):
    WT = W.T
    WihT = W_ih.T
    WhhT = W_hh.T
    b2 = b.reshape(1, D)
    bih2 = b_ih.reshape(1, 3 * D)
    bhh2 = b_hh.reshape(1, 3 * D)

    pad = E_PAD - E
    it = jnp.arange(pad, dtype=jnp.int32)
    src = jnp.concatenate([edge_index[0], (it * 97) % N])
    dst = jnp.concatenate([edge_index[1], N + (it % (ACC_ROWS - N))])
    src3 = (src.reshape(NCHUNK, NW, CHUNK).transpose(1, 0, 2)
            .reshape(NW * NWIN, WCH, CHUNK))
    dst3 = (dst.reshape(NCHUNK, NW, CHUNK).transpose(1, 0, 2)
            .reshape(NW * NWIN, WCH, CHUNK))

    h = x
    Wh = _linear(x, WT, b2)
    for step in range(STEPS):
        parts = _sc_scatter(Wh, src3, dst3)
        a0 = parts[:N]
        a1 = parts[ACC_ROWS:ACC_ROWS + N]
        if step + 1 < STEPS:
            h, Wh = _gru_step(a0, a1, h, WihT, WhhT, bih2, bhh2, WT, b2)
        else:
            h = _gru_last(a0, a1, h, WihT, WhhT, bih2, bhh2)
    return h


# speedup vs baseline: 1.0117x; 1.0036x over previous
import jax
import jax.numpy as jnp
from jax import lax
from jax.experimental import pallas as pl
from jax.experimental.pallas import tpu as pltpu
from jax.experimental.pallas import tpu_sc as plsc

N = 10000
E = 320000
D = 128
STEPS = 5

NC, NS = 2, 16
NW = NC * NS
CHUNK = 128
NCHUNK = 80
WCH = 8
NWIN = NCHUNK // WCH
E_PAD = NW * NCHUNK * CHUNK
ACC_ROWS = 10112
ROWS_PER_TILE = ACC_ROWS // NS


def _sc_body(wh_hbm, src_hbm, dst_hbm, out_hbm,
             sw0, dw0, sw1, dw1, rows_a, rows_b, acc_sh,
             gsa, gsb, ws0, ws1):
    c = lax.axis_index("c")
    s = lax.axis_index("s")
    wid = s * NC + c
    wbase = wid * NWIN

    rows = (rows_a, rows_b)
    gs = (gsa, gsb)

    def idx_load(w, sb, db, sem):
        pltpu.async_copy(src_hbm.at[wbase + w], sb, sem)
        pltpu.async_copy(dst_hbm.at[wbase + w], db, sem)

    def idx_wait(sb, db, sem):
        pltpu.make_async_copy(src_hbm.at[wbase], sb, sem).wait()
        pltpu.make_async_copy(dst_hbm.at[wbase], db, sem).wait()

    def gather(sb, j, par):
        pltpu.async_copy(wh_hbm.at[sb.at[j]], rows[par], gs[par])

    def gather_wait(par):
        pltpu.make_async_copy(wh_hbm.at[sw0.at[0]], rows[par], gs[par]).wait()

    def scat(db, j, par):
        pltpu.sync_copy(rows[par], acc_sh.at[db.at[j]], add=True)

    def process_window(sb, db, nsb, ndb, nws, prefetch_next):
        for j in range(WCH):
            par = j % 2
            gather_wait(par)
            if j + 1 < WCH:
                gather(sb, j + 1, 1 - par)
            elif prefetch_next:
                idx_wait(nsb, ndb, nws)
                gather(nsb, 0, 1 - par)
            scat(db, j, par)

    idx_load(0, sw0, dw0, ws0)
    idx_load(1, sw1, dw1, ws1)

    def zfill(i, _):
        r = i // (D // 16)
        k = i % (D // 16)
        rows_b[r, pl.ds(k * 16, 16)] = jnp.zeros((16,), jnp.float32)
        return 0
    lax.fori_loop(0, CHUNK * (D // 16), zfill, 0)
    base = s * ROWS_PER_TILE
    for j in range(ROWS_PER_TILE // CHUNK):
        pltpu.sync_copy(rows_b, acc_sh.at[pl.ds(base + j * CHUNK, CHUNK)])
    rem = ROWS_PER_TILE % CHUNK
    if rem:
        pltpu.sync_copy(
            rows_b.at[pl.ds(0, rem)],
            acc_sh.at[pl.ds(base + (ROWS_PER_TILE // CHUNK) * CHUNK, rem)])

    idx_wait(sw0, dw0, ws0)
    gather(sw0, 0, 0)
    plsc.subcore_barrier()

    def pair(p, _):
        process_window(sw0, dw0, sw1, dw1, ws1, True)
        idx_load(2 * p + 2, sw0, dw0, ws0)
        process_window(sw1, dw1, sw0, dw0, ws0, True)
        idx_load(2 * p + 3, sw1, dw1, ws1)
        return 0
    lax.fori_loop(0, NWIN // 2 - 1, pair, 0)

    process_window(sw0, dw0, sw1, dw1, ws1, True)
    process_window(sw1, dw1, None, None, None, False)
    plsc.subcore_barrier()

    pltpu.sync_copy(acc_sh.at[pl.ds(base, ROWS_PER_TILE)],
                    out_hbm.at[pl.ds(c * ACC_ROWS + base, ROWS_PER_TILE)])


_sc_scatter = pl.kernel(
    _sc_body,
    out_type=jax.ShapeDtypeStruct((NC * ACC_ROWS, D), jnp.float32),
    mesh=plsc.VectorSubcoreMesh(core_axis_name="c", subcore_axis_name="s"),
    scratch_types=[
        pltpu.VMEM((WCH, CHUNK), jnp.int32),
        pltpu.VMEM((WCH, CHUNK), jnp.int32),
        pltpu.VMEM((WCH, CHUNK), jnp.int32),
        pltpu.VMEM((WCH, CHUNK), jnp.int32),
        pltpu.VMEM((CHUNK, D), jnp.float32),
        pltpu.VMEM((CHUNK, D), jnp.float32),
        pltpu.VMEM_SHARED((ACC_ROWS, D), jnp.float32),
        pltpu.SemaphoreType.DMA,
        pltpu.SemaphoreType.DMA,
        pltpu.SemaphoreType.DMA,
        pltpu.SemaphoreType.DMA,
    ],
)


ROW_BLK = 1000


def _linear_body(x_ref, wT_ref, b_ref, out_ref):
    out_ref[...] = (
        jnp.dot(x_ref[...], wT_ref[...], preferred_element_type=jnp.float32)
        + b_ref[...]
    )


def _linear(x, WT, b2):
    return pl.pallas_call(
        _linear_body,
        grid=(N // ROW_BLK,),
        in_specs=[
            pl.BlockSpec((ROW_BLK, D), lambda i: (i, 0)),
            pl.BlockSpec((D, D), lambda i: (0, 0)),
            pl.BlockSpec((1, D), lambda i: (0, 0)),
        ],
        out_specs=pl.BlockSpec((ROW_BLK, D), lambda i: (i, 0)),
        out_shape=jax.ShapeDtypeStruct((N, D), jnp.float32),
    )(x, WT, b2)


def _gru_body(a0_ref, a1_ref, h_ref, wihT_ref, whhT_ref, bih_ref, bhh_ref,
              wT_ref, b_ref, h_out, wh_out):
    a = a0_ref[...] + a1_ref[...]
    h = h_ref[...]
    gi = jnp.dot(a, wihT_ref[...], preferred_element_type=jnp.float32) + bih_ref[...]
    gh = jnp.dot(h, whhT_ref[...], preferred_element_type=jnp.float32) + bhh_ref[...]
    r = jax.nn.sigmoid(gi[:, :D] + gh[:, :D])
    z = jax.nn.sigmoid(gi[:, D:2 * D] + gh[:, D:2 * D])
    n = jnp.tanh(gi[:, 2 * D:] + r * gh[:, 2 * D:])
    hn = (1.0 - z) * n + z * h
    h_out[...] = hn
    wh_out[...] = (
        jnp.dot(hn, wT_ref[...], preferred_element_type=jnp.float32) + b_ref[...]
    )


def _gru_last_body(a0_ref, a1_ref, h_ref, wihT_ref, whhT_ref, bih_ref,
                   bhh_ref, h_out):
    a = a0_ref[...] + a1_ref[...]
    h = h_ref[...]
    gi = jnp.dot(a, wihT_ref[...], preferred_element_type=jnp.float32) + bih_ref[...]
    gh = jnp.dot(h, whhT_ref[...], preferred_element_type=jnp.float32) + bhh_ref[...]
    r = jax.nn.sigmoid(gi[:, :D] + gh[:, :D])
    z = jax.nn.sigmoid(gi[:, D:2 * D] + gh[:, D:2 * D])
    n = jnp.tanh(gi[:, 2 * D:] + r * gh[:, 2 * D:])
    h_out[...] = (1.0 - z) * n + z * h


def _gru_last(a0, a1, h, WihT, WhhT, bih2, bhh2):
    return pl.pallas_call(
        _gru_last_body,
        grid=(N // ROW_BLK,),
        in_specs=[
            pl.BlockSpec((ROW_BLK, D), lambda i: (i, 0)),
            pl.BlockSpec((ROW_BLK, D), lambda i: (i, 0)),
            pl.BlockSpec((ROW_BLK, D), lambda i: (i, 0)),
            pl.BlockSpec((D, 3 * D), lambda i: (0, 0)),
            pl.BlockSpec((D, 3 * D), lambda i: (0, 0)),
            pl.BlockSpec((1, 3 * D), lambda i: (0, 0)),
            pl.BlockSpec((1, 3 * D), lambda i: (0, 0)),
        ],
        out_specs=pl.BlockSpec((ROW_BLK, D), lambda i: (i, 0)),
        out_shape=jax.ShapeDtypeStruct((N, D), jnp.float32),
    )(a0, a1, h, WihT, WhhT, bih2, bhh2)


def _gru_step(a0, a1, h, WihT, WhhT, bih2, bhh2, WT, b2):
    return pl.pallas_call(
        _gru_body,
        grid=(N // ROW_BLK,),
        in_specs=[
            pl.BlockSpec((ROW_BLK, D), lambda i: (i, 0)),
            pl.BlockSpec((ROW_BLK, D), lambda i: (i, 0)),
            pl.BlockSpec((ROW_BLK, D), lambda i: (i, 0)),
            pl.BlockSpec((D, 3 * D), lambda i: (0, 0)),
            pl.BlockSpec((D, 3 * D), lambda i: (0, 0)),
            pl.BlockSpec((1, 3 * D), lambda i: (0, 0)),
            pl.BlockSpec((1, 3 * D), lambda i: (0, 0)),
            pl.BlockSpec((D, D), lambda i: (0, 0)),
            pl.BlockSpec((1, D), lambda i: (0, 0)),
        ],
        out_specs=[
            pl.BlockSpec((ROW_BLK, D), lambda i: (i, 0)),
            pl.BlockSpec((ROW_BLK, D), lambda i: (i, 0)),
        ],
        out_shape=[
            jax.ShapeDtypeStruct((N, D), jnp.float32),
            jax.ShapeDtypeStruct((N, D), jnp.float32),
        ],
    )(a0, a1, h, WihT, WhhT, bih2, bhh2, WT, b2)


@jax.jit
def kernel(x, edge_index, W, b, W_ih, W_hh, b_ih, b_hh):
    WT = W.T
    WihT = W_ih.T
    WhhT = W_hh.T
    b2 = b.reshape(1, D)
    bih2 = b_ih.reshape(1, 3 * D)
    bhh2 = b_hh.reshape(1, 3 * D)

    pad = E_PAD - E
    it = jnp.arange(pad, dtype=jnp.int32)
    src = jnp.concatenate([edge_index[0], (it * 97) % N])
    dst = jnp.concatenate([edge_index[1], N + (it % (ACC_ROWS - N))])
    src3 = (src.reshape(NCHUNK, NW, CHUNK).transpose(1, 0, 2)
            .reshape(NW * NWIN, WCH, CHUNK))
    dst3 = (dst.reshape(NCHUNK, NW, CHUNK).transpose(1, 0, 2)
            .reshape(NW * NWIN, WCH, CHUNK))

    h = x
    Wh = _linear(x, WT, b2)
    for step in range(STEPS):
        parts = _sc_scatter(Wh, src3, dst3)
        a0 = parts[:N]
        a1 = parts[ACC_ROWS:ACC_ROWS + N]
        if step + 1 < STEPS:
            h, Wh = _gru_step(a0, a1, h, WihT, WhhT, bih2, bhh2, WT, b2)
        else:
            h = _gru_last(a0, a1, h, WihT, WhhT, bih2, bhh2)
    return h
